# baseline (speedup 1.0000x reference)
import functools

import jax
import jax.numpy as jnp
from jax.experimental import pallas as pl
from jax.experimental.pallas import tpu as pltpu
from jax.experimental.pallas import tpu_sc as plsc

IMG_SIZE = 128
UNO_RES = 64
KNN = 3
NF = 64
OC = 256
B = 2
N = IMG_SIZE * IMG_SIZE
Q = UNO_RES * UNO_RES
PAD_C = 16

QB = 128
NQB = Q // QB

QB2 = 512
NQB2 = Q // QB2

_BIG = 3.0e38


def _top3_body(qx_ref, qy_ref, cx_ref, cy_ref, idx_ref):
    g = pl.program_id(0)
    b = g // NQB
    qx = qx_ref[...]
    qy = qy_ref[...]
    cx = cx_ref[0]
    cy = cy_ref[0]
    qq = qx * qx + qy * qy
    cc = cx * cx + cy * cy
    q2 = jnp.concatenate([qx * -2.0, qy * -2.0], axis=1)
    c2 = jnp.concatenate([cx, cy], axis=0)
    cross = jnp.dot(q2, c2, preferred_element_type=jnp.float32)
    d2 = qq + cc + cross
    lane = jax.lax.broadcasted_iota(jnp.int32, (QB, N), 1)

    idxs = []
    for _ in range(KNN):
        m = jnp.min(d2, axis=1, keepdims=True)
        hit = d2 == m
        i = jnp.min(jnp.where(hit, lane, N), axis=1, keepdims=True)
        d2 = jnp.where(lane == i, _BIG, d2)
        idxs.append(i)

    fidx = jnp.concatenate(idxs, axis=1) + b * N
    idx_ref[0] = fidx


def _top3(qx, qy, cx, cy):
    grid = (B * NQB,)
    return pl.pallas_call(
        _top3_body,
        grid=grid,
        in_specs=[
            pl.BlockSpec((QB, 1), lambda g: (g % NQB, 0)),
            pl.BlockSpec((QB, 1), lambda g: (g % NQB, 0)),
            pl.BlockSpec((1, 1, N), lambda g: (g // NQB, 0, 0)),
            pl.BlockSpec((1, 1, N), lambda g: (g // NQB, 0, 0)),
        ],
        out_specs=pl.BlockSpec((1, QB, KNN), lambda g: (g, 0, 0)),
        out_shape=jax.ShapeDtypeStruct((B * NQB, QB, KNN), jnp.int32),
        compiler_params=pltpu.CompilerParams(
            dimension_semantics=("arbitrary",),
        ),
    )(qx, qy, cx, cy)


_NC = 2
_NS = 16
_NW = _NC * _NS
_ROWS = B * Q * KNN
_RPW = _ROWS // _NW

def _sc_gather_body(table_hbm, idx_hbm, out_hbm, idx_v, rows_v, sem):
    wid = jax.lax.axis_index("s") * _NC + jax.lax.axis_index("c")
    base = wid * _RPW
    pltpu.sync_copy(idx_hbm.at[pl.ds(base, _RPW)], idx_v)
    pltpu.async_copy(table_hbm.at[idx_v], rows_v, sem).wait()
    pltpu.sync_copy(rows_v, out_hbm.at[pl.ds(base, _RPW)])


@functools.cache
def _sc_gather_kernel():
    mesh = plsc.VectorSubcoreMesh(
        core_axis_name="c", subcore_axis_name="s",
        num_cores=_NC, num_subcores=_NS)
    return pl.kernel(
        _sc_gather_body,
        mesh=mesh,
        out_type=jax.ShapeDtypeStruct((_ROWS, PAD_C), jnp.float32),
        scratch_types=[
            pltpu.VMEM((_RPW,), jnp.int32),
            pltpu.VMEM((_RPW, PAD_C), jnp.float32),
            pltpu.SemaphoreType.DMA,
        ],
        compiler_params=pltpu.CompilerParams(use_tc_tiling_on_sc=False),
    )


def _sc_gather(table, idx):
    return _sc_gather_kernel()(table, idx)


def _trunk_body(xg0_ref, xg1_ref, xg2_ref, qx_ref, qy_ref,
                winp_ref, wuno_ref, buno_ref,
                wmid_ref, bmid_ref, wout_ref, bout_ref, out_ref, acc_ref):
    qb = pl.program_id(1)

    @pl.when(qb == 0)
    def _():
        acc_ref[...] = jnp.zeros_like(acc_ref)

    qx = qx_ref[...]
    qy = qy_ref[...]
    num = None
    s = None
    for ref in (xg0_ref, xg1_ref, xg2_ref):
        xgk = ref[0]
        dx = xgk[:, 3:4] - qx
        dy = xgk[:, 4:5] - qy
        sqd = dx * dx + dy * dy
        w = 1.0 / jnp.maximum(sqd, 1e-16)
        num = w * xgk if num is None else num + w * xgk
        s = w if s is None else s + w
    xc = num / s

    w2 = jnp.dot(winp_ref[...], wuno_ref[...],
                 preferred_element_type=jnp.float32)
    b2 = buno_ref[...]
    t = jnp.dot(xc, w2, preferred_element_type=jnp.float32) + b2
    u = jax.nn.gelu(jnp.dot(t, wmid_ref[...],
                            preferred_element_type=jnp.float32)
                    + bmid_ref[...])
    acc_ref[0:1, :] += jnp.sum(u, axis=0, keepdims=True)

    @pl.when(qb == NQB2 - 1)
    def _():
        pooled = acc_ref[0:1, :] * (1.0 / Q)
        res = (jnp.dot(pooled, wout_ref[...],
                       preferred_element_type=jnp.float32)
               + bout_ref[...])
        out_ref[...] = jnp.broadcast_to(res[None], (1, 8, OC))


def _trunk(xg, qx, qy, w_inp, w_uno, b2, w_mid, b_mid, w_out, b_out):
    grid = (B, NQB2)
    full = lambda i, j: (0, 0)
    out = pl.pallas_call(
        _trunk_body,
        grid=grid,
        in_specs=[
            pl.BlockSpec((1, QB2, PAD_C), lambda i, j: (0, i * NQB2 + j, 0)),
            pl.BlockSpec((1, QB2, PAD_C), lambda i, j: (1, i * NQB2 + j, 0)),
            pl.BlockSpec((1, QB2, PAD_C), lambda i, j: (2, i * NQB2 + j, 0)),
            pl.BlockSpec((QB2, 1), lambda i, j: (j, 0)),
            pl.BlockSpec((QB2, 1), lambda i, j: (j, 0)),
            pl.BlockSpec((PAD_C, NF), full),
            pl.BlockSpec((NF, NF), full),
            pl.BlockSpec((1, NF), full),
            pl.BlockSpec((NF, OC), full),
            pl.BlockSpec((1, OC), full),
            pl.BlockSpec((OC, OC), full),
            pl.BlockSpec((1, OC), full),
        ],
        out_specs=pl.BlockSpec((1, 8, OC), lambda i, j: (i, 0, 0)),
        out_shape=jax.ShapeDtypeStruct((B, 8, OC), jnp.float32),
        scratch_shapes=[pltpu.VMEM((8, OC), jnp.float32)],
        compiler_params=pltpu.CompilerParams(
            dimension_semantics=("arbitrary", "arbitrary"),
        ),
    )(xg, xg, xg, qx, qy, w_inp, w_uno, b2, w_mid, b_mid, w_out, b_out)
    return out[:, 0, :]


def kernel(x, coords, W_in, b_in, W_uno, b_uno, W_mid, b_mid, W_out, b_out):
    lin = jnp.linspace(0.0, 1.0, UNO_RES)
    qx = jnp.repeat(lin, UNO_RES).reshape(Q, 1)
    qy = jnp.tile(lin, UNO_RES).reshape(Q, 1)
    cx = coords[:, :, 0].reshape(B, 1, N)
    cy = coords[:, :, 1].reshape(B, 1, N)
    xp = jnp.pad(
        jnp.concatenate([x.reshape(B * N, 3), coords.reshape(B * N, 2)], axis=1),
        ((0, 0), (0, PAD_C - 5)))
    w_inp = jnp.pad(W_in, ((0, PAD_C - 3), (0, 0)))
    b2 = (b_in @ W_uno + b_uno).reshape(1, NF)
    bmid2 = b_mid.reshape(1, OC)
    bout2 = b_out.reshape(1, OC)

    idx = _top3(qx, qy, cx, cy)
    idx_flat = idx.reshape(B * Q, KNN).T.reshape(_ROWS)

    xg = _sc_gather(xp, idx_flat)
    xg3 = xg.reshape(KNN, B * Q, PAD_C)

    return _trunk(xg3, qx, qy, w_inp, W_uno, b2, W_mid, bmid2, W_out, bout2)


# speedup vs baseline: 1.0542x; 1.0542x over previous
import functools

import jax
import jax.numpy as jnp
from jax.experimental import pallas as pl
from jax.experimental.pallas import tpu as pltpu
from jax.experimental.pallas import tpu_sc as plsc

IMG_SIZE = 128
UNO_RES = 64
KNN = 3
NF = 64
OC = 256
B = 2
N = IMG_SIZE * IMG_SIZE
Q = UNO_RES * UNO_RES
PAD_C = 16

QB = 128
NQB = Q // QB

QB2 = 512
NQB2 = Q // QB2

_BIG = 3.0e38


def _top3_body(qx_ref, qy_ref, cx_ref, cy_ref, idx_ref):
    g = pl.program_id(0)
    b = g // NQB
    qx = qx_ref[...]
    qy = qy_ref[...]
    cx = cx_ref[0]
    cy = cy_ref[0]
    qq = qx * qx + qy * qy
    cc = cx * cx + cy * cy
    q2 = jnp.concatenate([qx, qy], axis=1)
    c2 = jnp.concatenate([cx, cy], axis=0)
    cross = jnp.dot(q2, c2, preferred_element_type=jnp.float32)
    d2 = qq + cc - 2.0 * cross
    lane = jax.lax.broadcasted_iota(jnp.int32, (QB, N), 1)

    idxs = []
    for _ in range(KNN):
        m = jnp.min(d2, axis=1, keepdims=True)
        hit = d2 == m
        i = jnp.min(jnp.where(hit, lane, N), axis=1, keepdims=True)
        d2 = jnp.where(lane == i, _BIG, d2)
        idxs.append(i)

    fidx = jnp.concatenate(idxs, axis=1) + b * N
    idx_ref[0] = fidx


def _top3(qx, qy, cx, cy):
    grid = (B * NQB,)
    return pl.pallas_call(
        _top3_body,
        grid=grid,
        in_specs=[
            pl.BlockSpec((QB, 1), lambda g: (g % NQB, 0)),
            pl.BlockSpec((QB, 1), lambda g: (g % NQB, 0)),
            pl.BlockSpec((1, 1, N), lambda g: (g // NQB, 0, 0)),
            pl.BlockSpec((1, 1, N), lambda g: (g // NQB, 0, 0)),
        ],
        out_specs=pl.BlockSpec((1, QB, KNN), lambda g: (g, 0, 0)),
        out_shape=jax.ShapeDtypeStruct((B * NQB, QB, KNN), jnp.int32),
        compiler_params=pltpu.CompilerParams(
            dimension_semantics=("arbitrary",),
        ),
    )(qx, qy, cx, cy)


_NC = 2
_NS = 16
_NW = _NC * _NS
_ROWS = B * Q * KNN
_RPW = _ROWS // _NW

def _sc_gather_body(table_hbm, idx_hbm, out_hbm, idx_v, rows_v, sem):
    wid = jax.lax.axis_index("s") * _NC + jax.lax.axis_index("c")
    base = wid * _RPW
    pltpu.sync_copy(idx_hbm.at[pl.ds(base, _RPW)], idx_v)
    pltpu.async_copy(table_hbm.at[idx_v], rows_v, sem).wait()
    pltpu.sync_copy(rows_v, out_hbm.at[pl.ds(base, _RPW)])


@functools.cache
def _sc_gather_kernel():
    mesh = plsc.VectorSubcoreMesh(
        core_axis_name="c", subcore_axis_name="s",
        num_cores=_NC, num_subcores=_NS)
    return pl.kernel(
        _sc_gather_body,
        mesh=mesh,
        out_type=jax.ShapeDtypeStruct((_ROWS, PAD_C), jnp.float32),
        scratch_types=[
            pltpu.VMEM((_RPW,), jnp.int32),
            pltpu.VMEM((_RPW, PAD_C), jnp.float32),
            pltpu.SemaphoreType.DMA,
        ],
        compiler_params=pltpu.CompilerParams(use_tc_tiling_on_sc=False),
    )


def _sc_gather(table, idx):
    return _sc_gather_kernel()(table, idx)


def _trunk_body(xg0_ref, xg1_ref, xg2_ref, qx_ref, qy_ref,
                winp_ref, wuno_ref, buno_ref,
                wmid_ref, bmid_ref, wout_ref, bout_ref, out_ref, acc_ref):
    qb = pl.program_id(1)

    @pl.when(qb == 0)
    def _():
        acc_ref[...] = jnp.zeros_like(acc_ref)

    qx = qx_ref[...]
    qy = qy_ref[...]
    num = None
    s = None
    for ref in (xg0_ref, xg1_ref, xg2_ref):
        xgk = ref[0]
        dx = xgk[:, 3:4] - qx
        dy = xgk[:, 4:5] - qy
        sqd = dx * dx + dy * dy
        w = 1.0 / jnp.maximum(sqd, 1e-16)
        num = w * xgk if num is None else num + w * xgk
        s = w if s is None else s + w
    xc = num / s

    w2 = jnp.dot(winp_ref[...], wuno_ref[...],
                 preferred_element_type=jnp.float32)
    b2 = buno_ref[...]
    t = jnp.dot(xc, w2, preferred_element_type=jnp.float32) + b2
    u = jax.nn.gelu(jnp.dot(t, wmid_ref[...],
                            preferred_element_type=jnp.float32)
                    + bmid_ref[...])
    acc_ref[0:1, :] += jnp.sum(u, axis=0, keepdims=True)

    @pl.when(qb == NQB2 - 1)
    def _():
        pooled = acc_ref[0:1, :] * (1.0 / Q)
        res = (jnp.dot(pooled, wout_ref[...],
                       preferred_element_type=jnp.float32)
               + bout_ref[...])
        out_ref[...] = jnp.broadcast_to(res[None], (1, 8, OC))


def _trunk(xg, qx, qy, w_inp, w_uno, b2, w_mid, b_mid, w_out, b_out):
    grid = (B, NQB2)
    full = lambda i, j: (0, 0)
    out = pl.pallas_call(
        _trunk_body,
        grid=grid,
        in_specs=[
            pl.BlockSpec((1, QB2, PAD_C), lambda i, j: (0, i * NQB2 + j, 0)),
            pl.BlockSpec((1, QB2, PAD_C), lambda i, j: (1, i * NQB2 + j, 0)),
            pl.BlockSpec((1, QB2, PAD_C), lambda i, j: (2, i * NQB2 + j, 0)),
            pl.BlockSpec((QB2, 1), lambda i, j: (j, 0)),
            pl.BlockSpec((QB2, 1), lambda i, j: (j, 0)),
            pl.BlockSpec((PAD_C, NF), full),
            pl.BlockSpec((NF, NF), full),
            pl.BlockSpec((1, NF), full),
            pl.BlockSpec((NF, OC), full),
            pl.BlockSpec((1, OC), full),
            pl.BlockSpec((OC, OC), full),
            pl.BlockSpec((1, OC), full),
        ],
        out_specs=pl.BlockSpec((1, 8, OC), lambda i, j: (i, 0, 0)),
        out_shape=jax.ShapeDtypeStruct((B, 8, OC), jnp.float32),
        scratch_shapes=[pltpu.VMEM((8, OC), jnp.float32)],
        compiler_params=pltpu.CompilerParams(
            dimension_semantics=("arbitrary", "arbitrary"),
        ),
    )(xg, xg, xg, qx, qy, w_inp, w_uno, b2, w_mid, b_mid, w_out, b_out)
    return out[:, 0, :]


def kernel(x, coords, W_in, b_in, W_uno, b_uno, W_mid, b_mid, W_out, b_out):
    lin = jnp.linspace(0.0, 1.0, UNO_RES)
    qx = jnp.repeat(lin, UNO_RES).reshape(Q, 1)
    qy = jnp.tile(lin, UNO_RES).reshape(Q, 1)
    cx = coords[:, :, 0].reshape(B, 1, N)
    cy = coords[:, :, 1].reshape(B, 1, N)
    xp = jnp.pad(
        jnp.concatenate([x.reshape(B * N, 3), coords.reshape(B * N, 2)], axis=1),
        ((0, 0), (0, PAD_C - 5)))
    w_inp = jnp.pad(W_in, ((0, PAD_C - 3), (0, 0)))
    b2 = (b_in @ W_uno + b_uno).reshape(1, NF)
    bmid2 = b_mid.reshape(1, OC)
    bout2 = b_out.reshape(1, OC)

    idx = _top3(qx, qy, cx, cy)
    idx_flat = idx.reshape(B * Q, KNN).T.reshape(_ROWS)

    xg = _sc_gather(xp, idx_flat)
    xg3 = xg.reshape(KNN, B * Q, PAD_C)

    return _trunk(xg3, qx, qy, w_inp, W_uno, b2, W_mid, bmid2, W_out, bout2)
